# baseline (speedup 1.0000x reference)
import jax
from jax import lax
import jax.numpy as jnp
from jax.experimental import pallas as pl
from jax.experimental.pallas import tpu as pltpu
from jax.experimental.pallas import tpu_sc as plsc

_EMBED = 64
_MODEL = 256
_SCALE = 16.0

_NC = 2
_NS = 16
_NW = _NC * _NS
_CHUNK = 128

_M_TILE = 2048
_PIPE = 4


def _sc_gather(table2, idx2):
    n = idx2.shape[0]
    b_per_w = n // _NW
    n_chunks = b_per_w // _CHUNK
    assert n_chunks % 2 == 0
    mesh = plsc.VectorSubcoreMesh(core_axis_name="c", subcore_axis_name="s")

    @pl.kernel(
        out_type=jax.ShapeDtypeStruct((n, 2 * _EMBED), table2.dtype),
        mesh=mesh,
        scratch_types=[
            pltpu.VMEM((b_per_w,), jnp.int32),
            pltpu.VMEM((_CHUNK, 2 * _EMBED), table2.dtype),
            pltpu.VMEM((_CHUNK, 2 * _EMBED), table2.dtype),
            pltpu.SemaphoreType.DMA,
            pltpu.SemaphoreType.DMA,
        ],
    )
    def gather_kernel(table_hbm, idx_hbm, out_hbm, idx_v, r0, r1, s0, s1):
        wid = lax.axis_index("s") * _NC + lax.axis_index("c")
        wbase = wid * b_per_w

        def start(j, rows, sem):
            pltpu.async_copy(
                table_hbm.at[idx_v.at[pl.ds(j * _CHUNK, _CHUNK)]], rows, sem
            )

        def drain(j, rows, sem):
            pltpu.make_async_copy(
                table_hbm.at[idx_v.at[pl.ds(j * _CHUNK, _CHUNK)]], rows, sem
            ).wait()

        pltpu.sync_copy(idx_hbm.at[pl.ds(wbase, b_per_w)], idx_v)
        start(0, r0, s0)

        @pl.loop(0, n_chunks // 2)
        def _(jj):
            j = jj * 2
            start(j + 1, r1, s1)
            drain(j, r0, s0)
            pltpu.sync_copy(r0, out_hbm.at[pl.ds(wbase + j * _CHUNK, _CHUNK)])

            @pl.when(j + 2 < n_chunks)
            def _():
                start(j + 2, r0, s0)

            drain(j + 1, r1, s1)
            pltpu.sync_copy(r1, out_hbm.at[pl.ds(wbase + (j + 1) * _CHUNK, _CHUNK)])

    return gather_kernel(table2, idx2)


def _mm_body(emb_hbm, p_ref, w_ref, b_ref, o_ref, abuf, s0, s1):
    i = pl.program_id(0)
    nt = pl.num_programs(0)
    slot = lax.rem(i, 2)

    def start(tile, s, sem):
        pltpu.make_async_copy(
            emb_hbm.at[pl.ds(tile * _M_TILE, _M_TILE)], abuf.at[s], sem
        ).start()

    def wait(tile, s, sem):
        pltpu.make_async_copy(
            emb_hbm.at[pl.ds(tile * _M_TILE, _M_TILE)], abuf.at[s], sem
        ).wait()

    @pl.when(i == 0)
    def _():
        start(0, 0, s0)

    @pl.when(jnp.logical_and(i + 1 < nt, slot == 0))
    def _():
        start(i + 1, 1, s1)

    @pl.when(jnp.logical_and(i + 1 < nt, slot == 1))
    def _():
        start(i + 1, 0, s0)

    @pl.when(slot == 0)
    def _():
        wait(i, 0, s0)

    @pl.when(slot == 1)
    def _():
        wait(i, 1, s1)

    a = abuf[slot]
    half = lax.broadcasted_iota(jnp.int32, (_M_TILE, 2 * _EMBED), 1) // _EMBED
    mask = (half == p_ref[...]).astype(jnp.float32)
    acc = jax.lax.dot_general(
        a * mask, w_ref[...], (((1,), (0,)), ((), ())),
        preferred_element_type=jnp.float32,
    )
    o_ref[...] = (acc + b_ref[...]) * _SCALE


def _tc_project_chunk(dst, emb, par, w2, b2d, c, n_total):
    nc_rows = emb.shape[0]
    base = c * (nc_rows // _M_TILE)
    out_spec = pl.BlockSpec((_M_TILE, _MODEL), lambda i: (i + base, 0))
    out_shape = jax.ShapeDtypeStruct((n_total, _MODEL), jnp.float32)
    data_specs = [
        pl.BlockSpec(memory_space=pltpu.MemorySpace.HBM),
        pl.BlockSpec((_M_TILE, 1), lambda i: (i, 0)),
        pl.BlockSpec((2 * _EMBED, _MODEL), lambda i: (0, 0)),
        pl.BlockSpec((1, _MODEL), lambda i: (0, 0)),
    ]
    scratch = [
        pltpu.VMEM((2, _M_TILE, 2 * _EMBED), jnp.float32),
        pltpu.SemaphoreType.DMA,
        pltpu.SemaphoreType.DMA,
    ]
    if dst is None:
        return pl.pallas_call(
            _mm_body,
            grid=(nc_rows // _M_TILE,),
            in_specs=data_specs,
            out_specs=out_spec,
            out_shape=out_shape,
            scratch_shapes=scratch,
            compiler_params=pltpu.CompilerParams(
                dimension_semantics=("arbitrary",)
            ),
        )(emb, par, w2, b2d)

    def body(a_ref, p_ref, w_ref, b_ref, dst_ref, o_ref, abuf, s0, s1):
        del dst_ref
        _mm_body(a_ref, p_ref, w_ref, b_ref, o_ref, abuf, s0, s1)

    return pl.pallas_call(
        body,
        grid=(nc_rows // _M_TILE,),
        in_specs=data_specs + [pl.BlockSpec(memory_space=pltpu.MemorySpace.HBM)],
        out_specs=out_spec,
        out_shape=out_shape,
        scratch_shapes=scratch,
        input_output_aliases={4: 0},
        compiler_params=pltpu.CompilerParams(dimension_semantics=("arbitrary",)),
    )(emb, par, w2, b2d, dst)


def kernel(x, table, W, b):
    bsz, seq = x.shape
    n = bsz * seq
    idx = x.reshape(n).astype(jnp.int32)
    table2 = table.reshape(table.shape[0] // 2, 2 * _EMBED)
    w2 = jnp.concatenate([W.T, W.T], axis=0)
    b2d = b.reshape(1, _MODEL)

    nc_rows = n // _PIPE
    out = None
    for c in range(_PIPE):
        idx_c = lax.dynamic_slice_in_dim(idx, c * nc_rows, nc_rows)
        emb_c = _sc_gather(table2, idx_c >> 1)
        par_c = (idx_c & 1).reshape(-1, 1)
        out = _tc_project_chunk(out, emb_c, par_c, w2, b2d, c, n)
    return out.reshape(bsz, seq, _MODEL)


# speedup vs baseline: 1.1774x; 1.1774x over previous
import jax
from jax import lax
import jax.numpy as jnp
from jax.experimental import pallas as pl
from jax.experimental.pallas import tpu as pltpu
from jax.experimental.pallas import tpu_sc as plsc

_EMBED = 64
_MODEL = 256
_SCALE = 16.0

_NC = 2
_NS = 16
_NW = _NC * _NS
_CHUNK = 128

_M_TILE = 2048
_PIPE = 4


def _sc_gather(table2, idx2):
    n = idx2.shape[0]
    b_per_w = n // _NW
    n_chunks = b_per_w // _CHUNK
    assert n_chunks % 2 == 0
    mesh = plsc.VectorSubcoreMesh(core_axis_name="c", subcore_axis_name="s")

    @pl.kernel(
        out_type=jax.ShapeDtypeStruct((n, 2 * _EMBED), table2.dtype),
        mesh=mesh,
        scratch_types=[
            pltpu.VMEM((b_per_w,), jnp.int32),
            pltpu.VMEM((_CHUNK, 2 * _EMBED), table2.dtype),
            pltpu.VMEM((_CHUNK, 2 * _EMBED), table2.dtype),
            pltpu.SemaphoreType.DMA,
            pltpu.SemaphoreType.DMA,
        ],
    )
    def gather_kernel(table_hbm, idx_hbm, out_hbm, idx_v, r0, r1, s0, s1):
        wid = lax.axis_index("s") * _NC + lax.axis_index("c")
        wbase = wid * b_per_w

        def start(j, rows, sem):
            pltpu.async_copy(
                table_hbm.at[idx_v.at[pl.ds(j * _CHUNK, _CHUNK)]], rows, sem
            )

        def drain(j, rows, sem):
            pltpu.make_async_copy(
                table_hbm.at[idx_v.at[pl.ds(j * _CHUNK, _CHUNK)]], rows, sem
            ).wait()

        pltpu.sync_copy(idx_hbm.at[pl.ds(wbase, b_per_w)], idx_v)
        start(0, r0, s0)

        @pl.loop(0, n_chunks // 2)
        def _(jj):
            j = jj * 2
            start(j + 1, r1, s1)
            drain(j, r0, s0)
            pltpu.sync_copy(r0, out_hbm.at[pl.ds(wbase + j * _CHUNK, _CHUNK)])

            @pl.when(j + 2 < n_chunks)
            def _():
                start(j + 2, r0, s0)

            drain(j + 1, r1, s1)
            pltpu.sync_copy(r1, out_hbm.at[pl.ds(wbase + (j + 1) * _CHUNK, _CHUNK)])

    return gather_kernel(table2, idx2)


def _mm_body(emb_hbm, p_ref, w_ref, b_ref, o_ref, abuf, s0, s1):
    i = pl.program_id(0)
    nt = pl.num_programs(0)
    slot = lax.rem(i, 2)

    def start(tile, s, sem):
        pltpu.make_async_copy(
            emb_hbm.at[pl.ds(tile * _M_TILE, _M_TILE)], abuf.at[s], sem
        ).start()

    def wait(tile, s, sem):
        pltpu.make_async_copy(
            emb_hbm.at[pl.ds(tile * _M_TILE, _M_TILE)], abuf.at[s], sem
        ).wait()

    @pl.when(i == 0)
    def _():
        start(0, 0, s0)

    @pl.when(jnp.logical_and(i + 1 < nt, slot == 0))
    def _():
        start(i + 1, 1, s1)

    @pl.when(jnp.logical_and(i + 1 < nt, slot == 1))
    def _():
        start(i + 1, 0, s0)

    @pl.when(slot == 0)
    def _():
        wait(i, 0, s0)

    @pl.when(slot == 1)
    def _():
        wait(i, 1, s1)

    a = abuf[slot]
    nb = _M_TILE // 128
    p2 = (p_ref[...] & 1).astype(jnp.float32)
    bsel = (
        lax.broadcasted_iota(jnp.int32, (_M_TILE, nb), 0) // 128
        == lax.broadcasted_iota(jnp.int32, (_M_TILE, nb), 1)
    ).astype(jnp.float32)
    rep = jax.lax.dot_general(
        bsel, p2, (((1,), (0,)), ((), ())), preferred_element_type=jnp.float32
    )
    delta = (
        lax.broadcasted_iota(jnp.int32, (_M_TILE, 128), 0) % 128
        == lax.broadcasted_iota(jnp.int32, (_M_TILE, 128), 1)
    )
    pcol = jnp.sum(jnp.where(delta, rep, 0.0), axis=1, keepdims=True)
    half = lax.broadcasted_iota(jnp.int32, (_M_TILE, 2 * _EMBED), 1) // _EMBED
    a_m = jnp.where(half.astype(jnp.float32) == pcol, a, jnp.float32(0))
    acc = jax.lax.dot_general(
        a_m, w_ref[...], (((1,), (0,)), ((), ())),
        preferred_element_type=jnp.float32,
    )
    o_ref[...] = (acc + b_ref[...]) * _SCALE


def _tc_project_chunk(dst, emb, par, w2, b2d, c, n_total):
    nc_rows = emb.shape[0]
    base = c * (nc_rows // _M_TILE)
    out_spec = pl.BlockSpec((_M_TILE, _MODEL), lambda i: (i + base, 0))
    out_shape = jax.ShapeDtypeStruct((n_total, _MODEL), jnp.float32)
    data_specs = [
        pl.BlockSpec(memory_space=pltpu.MemorySpace.HBM),
        pl.BlockSpec((_M_TILE // 128, 128), lambda i: (i, 0)),
        pl.BlockSpec((2 * _EMBED, _MODEL), lambda i: (0, 0)),
        pl.BlockSpec((1, _MODEL), lambda i: (0, 0)),
    ]
    scratch = [
        pltpu.VMEM((2, _M_TILE, 2 * _EMBED), jnp.float32),
        pltpu.SemaphoreType.DMA,
        pltpu.SemaphoreType.DMA,
    ]
    if dst is None:
        return pl.pallas_call(
            _mm_body,
            grid=(nc_rows // _M_TILE,),
            in_specs=data_specs,
            out_specs=out_spec,
            out_shape=out_shape,
            scratch_shapes=scratch,
            compiler_params=pltpu.CompilerParams(
                dimension_semantics=("arbitrary",)
            ),
        )(emb, par, w2, b2d)

    def body(a_ref, p_ref, w_ref, b_ref, dst_ref, o_ref, abuf, s0, s1):
        del dst_ref
        _mm_body(a_ref, p_ref, w_ref, b_ref, o_ref, abuf, s0, s1)

    return pl.pallas_call(
        body,
        grid=(nc_rows // _M_TILE,),
        in_specs=data_specs + [pl.BlockSpec(memory_space=pltpu.MemorySpace.HBM)],
        out_specs=out_spec,
        out_shape=out_shape,
        scratch_shapes=scratch,
        input_output_aliases={4: 0},
        compiler_params=pltpu.CompilerParams(dimension_semantics=("arbitrary",)),
    )(emb, par, w2, b2d, dst)


def kernel(x, table, W, b):
    bsz, seq = x.shape
    n = bsz * seq
    idx = x.reshape(n).astype(jnp.int32)
    table2 = table.reshape(table.shape[0] // 2, 2 * _EMBED)
    w2 = jnp.concatenate([W.T, W.T], axis=0)
    b2d = b.reshape(1, _MODEL)

    nc_rows = n // _PIPE
    out = None
    for c in range(_PIPE):
        idx_c = lax.dynamic_slice_in_dim(idx, c * nc_rows, nc_rows)
        emb_c = _sc_gather(table2, idx_c >> 1)
        idx2d_c = idx_c.reshape(nc_rows // 128, 128)
        out = _tc_project_chunk(out, emb_c, idx2d_c, w2, b2d, c, n)
    return out.reshape(bsz, seq, _MODEL)
